# baseline (speedup 1.0000x reference)
import functools

import jax
import jax.numpy as jnp
from jax.experimental import pallas as pl
from jax.experimental.pallas import tpu as pltpu

_DECAY = 0.1
_EPS = 1e-8
_LN_EPS = 1e-5


def _tc_body(lens_ref, h_ref, scale_ref, bias_ref, out_ref, *, seg_per_blk, T, D):
    i = pl.program_id(0)
    t_col = jax.lax.broadcasted_iota(jnp.int32, (T, 1), 0).astype(jnp.float32)
    w_base = jnp.exp(-_DECAY * ((T - 1) - t_col))
    scale = scale_ref[0, :]
    bias = bias_ref[0, :]
    for j in range(seg_per_blk):
        L = lens_ref[i * seg_per_blk + j]
        Lf = L.astype(jnp.float32)
        mask = (t_col < Lf).astype(jnp.float32)
        w = w_base * mask
        wsum = jnp.maximum(jnp.sum(w), _EPS)
        e = jnp.sum(h_ref[j] * w, axis=0) / wsum
        mu = jnp.mean(e)
        var = jnp.mean((e - mu) ** 2)
        e_ln = (e - mu) * jax.lax.rsqrt(var + _LN_EPS) * scale + bias
        out_ref[j, :] = jnp.where(L >= 1, e_ln, e)


def kernel(H, valid_lens, ln_scale, ln_bias):
    B, F, T, D = H.shape
    S = B * F
    seg_per_blk = 8
    grid = (S // seg_per_blk,)
    H2 = H.reshape(S, T, D)
    lens = valid_lens.reshape(S).astype(jnp.int32)
    scale2 = ln_scale.reshape(1, D)
    bias2 = ln_bias.reshape(1, D)

    out = pl.pallas_call(
        functools.partial(_tc_body, seg_per_blk=seg_per_blk, T=T, D=D),
        grid_spec=pltpu.PrefetchScalarGridSpec(
            num_scalar_prefetch=1,
            grid=grid,
            in_specs=[
                pl.BlockSpec((seg_per_blk, T, D), lambda i, lens: (i, 0, 0)),
                pl.BlockSpec((1, D), lambda i, lens: (0, 0)),
                pl.BlockSpec((1, D), lambda i, lens: (0, 0)),
            ],
            out_specs=pl.BlockSpec((seg_per_blk, D), lambda i, lens: (i, 0)),
        ),
        out_shape=jax.ShapeDtypeStruct((S, D), jnp.float32),
    )(lens, H2, scale2, bias2)
    return out.reshape(B, F, D)


# speedup vs baseline: 1.9989x; 1.9989x over previous
import functools

import jax
import jax.numpy as jnp
from jax.experimental import pallas as pl
from jax.experimental.pallas import tpu as pltpu

_DECAY = 0.1
_EPS = 1e-8
_LN_EPS = 1e-5


def _tc_body(h_ref, lens_ref, scale_ref, bias_ref, out_ref, *, SEG, T, D):
    lens_f = lens_ref[...].astype(jnp.float32)
    col = jax.lax.broadcasted_iota(jnp.int32, (SEG, SEG * T), 1)
    srow = jax.lax.broadcasted_iota(jnp.int32, (SEG, SEG * T), 0)
    t = col % T
    sp = col // T
    w_base = jnp.exp(-_DECAY * ((T - 1) - t).astype(jnp.float32))
    valid = (t.astype(jnp.float32) < lens_f) & (sp == srow)
    wbd = jnp.where(valid, w_base, 0.0)
    h = h_ref[...].reshape(SEG * T, D)
    e = jax.lax.dot_general(
        wbd, h, (((1,), (0,)), ((), ())), preferred_element_type=jnp.float32
    )
    r = jnp.exp(jnp.float32(_DECAY))
    wsum = jnp.exp(-_DECAY * (T - 1)) * (jnp.exp(_DECAY * lens_f) - 1.0) / (r - 1.0)
    wsum = jnp.maximum(wsum, _EPS)
    e = e / wsum
    mu = jnp.mean(e, axis=1, keepdims=True)
    var = jnp.mean((e - mu) ** 2, axis=1, keepdims=True)
    e_ln = (e - mu) * jax.lax.rsqrt(var + _LN_EPS) * scale_ref[...] + bias_ref[...]
    out_ref[...] = jnp.where(lens_f >= 1.0, e_ln, e)


def kernel(H, valid_lens, ln_scale, ln_bias):
    B, F, T, D = H.shape
    S = B * F
    SEG = 16
    grid = (S // SEG,)
    H2 = H.reshape(S, T, D)
    lens2 = valid_lens.reshape(S, 1).astype(jnp.int32)
    scale2 = ln_scale.reshape(1, D)
    bias2 = ln_bias.reshape(1, D)

    out = pl.pallas_call(
        functools.partial(_tc_body, SEG=SEG, T=T, D=D),
        grid=grid,
        in_specs=[
            pl.BlockSpec((SEG, T, D), lambda i: (i, 0, 0)),
            pl.BlockSpec((SEG, 1), lambda i: (i, 0)),
            pl.BlockSpec((1, D), lambda i: (0, 0)),
            pl.BlockSpec((1, D), lambda i: (0, 0)),
        ],
        out_specs=pl.BlockSpec((SEG, D), lambda i: (i, 0)),
        out_shape=jax.ShapeDtypeStruct((S, D), jnp.float32),
    )(H2, lens2, scale2, bias2)
    return out.reshape(B, F, D)
